# baseline (speedup 1.0000x reference)
import functools

import jax
import jax.numpy as jnp
from jax import lax
from jax.experimental import pallas as pl
from jax.experimental.pallas import tpu as pltpu
from jax.experimental.pallas import tpu_sc as plsc

NBINS_X = 1000
MAX_Y_TOKENS = 201
EMBED_DIM = 64
MAX_LEN = 200
B = 4096
T = 200
N = B * T

CHUNK = 200


def _make_kernel():
    info = plsc.get_sparse_core_info()
    nc, ns = info.num_cores, info.num_subcores
    nw = nc * ns
    n_chunks = B // nw
    assert B % nw == 0 and n_chunks % 2 == 0 and n_chunks >= 6

    mesh = plsc.VectorSubcoreMesh(core_axis_name="c", subcore_axis_name="s")

    f32 = jnp.float32
    i32 = jnp.int32

    @functools.partial(
        pl.kernel,
        mesh=mesh,
        out_type=jax.ShapeDtypeStruct((B, T, EMBED_DIM), f32),
        compiler_params=pltpu.CompilerParams(use_tc_tiling_on_sc=False),
        scratch_types=[
            pltpu.VMEM((CHUNK,), i32), pltpu.VMEM((CHUNK,), i32),
            pltpu.VMEM((CHUNK,), i32), pltpu.VMEM((CHUNK,), i32),
            pltpu.VMEM((CHUNK, EMBED_DIM), f32), pltpu.VMEM((CHUNK, EMBED_DIM), f32),
            pltpu.VMEM((CHUNK, EMBED_DIM), f32), pltpu.VMEM((CHUNK, EMBED_DIM), f32),
            pltpu.VMEM((CHUNK, EMBED_DIM), f32), pltpu.VMEM((CHUNK, EMBED_DIM), f32),
            pltpu.VMEM((CHUNK, EMBED_DIM), f32),
            pltpu.SemaphoreType.DMA, pltpu.SemaphoreType.DMA,
            pltpu.SemaphoreType.DMA, pltpu.SemaphoreType.DMA,
            pltpu.SemaphoreType.DMA, pltpu.SemaphoreType.DMA,
            pltpu.SemaphoreType.DMA, pltpu.SemaphoreType.DMA,
        ],
    )
    def k(x_tok, y_tok, x_table, y_table, pos_table, out,
          ix_a, ix_b, iy_a, iy_b, xr_a, xr_b, yr_a, yr_b, acc_a, acc_b,
          pos_v, sgx_a, sgx_b, sgy_a, sgy_b, so_a, so_b, si_a, si_b):
        wid = lax.axis_index("s") * nc + lax.axis_index("c")
        w_row = wid * n_chunks

        ix = (ix_a, ix_b)
        iy = (iy_a, iy_b)
        xr = (xr_a, xr_b)
        yr = (yr_a, yr_b)
        acc = (acc_a, acc_b)
        sgx = (sgx_a, sgx_b)
        sgy = (sgy_a, sgy_b)
        so = (so_a, so_b)
        si = (si_a, si_b)

        pltpu.sync_copy(pos_table, pos_v)

        def start_gathers(p):
            pltpu.async_copy(x_table.at[ix[p]], xr[p], sgx[p])
            pltpu.async_copy(y_table.at[iy[p]], yr[p], sgy[p])

        def wait_gathers(p):
            pltpu.make_async_copy(x_table.at[ix[p]], xr[p], sgx[p]).wait()
            pltpu.make_async_copy(y_table.at[iy[p]], yr[p], sgy[p]).wait()

        def start_idx_load(row, p):
            pltpu.async_copy(x_tok.at[row], ix[p], si[p])
            pltpu.async_copy(y_tok.at[row], iy[p], si[p])

        def wait_idx_load(p):
            pltpu.make_async_copy(x_tok.at[0], ix[p], si[p]).wait()
            pltpu.make_async_copy(y_tok.at[0], iy[p], si[p]).wait()

        def compute(p):
            xp, yp, ap = xr[p], yr[p], acc[p]
            nj = EMBED_DIM // 16

            def add_row(i, c):
                xs = [xp[i, pl.ds(j * 16, 16)] for j in range(nj)]
                ys = [yp[i, pl.ds(j * 16, 16)] for j in range(nj)]
                ps = [pos_v[i, pl.ds(j * 16, 16)] for j in range(nj)]
                ss = [(xs[j] + ys[j]) + ps[j] for j in range(nj)]
                for j in range(nj):
                    ap[i, pl.ds(j * 16, 16)] = ss[j]
                return c

            lax.fori_loop(0, CHUNK, add_row, 0, unroll=2)

        def start_out(row, p):
            pltpu.async_copy(acc[p], out.at[row], so[p])

        def wait_out(p):
            pltpu.make_async_copy(acc[p], out.at[0], so[p]).wait()

        pltpu.sync_copy(x_tok.at[w_row], ix_a)
        pltpu.sync_copy(y_tok.at[w_row], iy_a)
        start_gathers(0)
        pltpu.sync_copy(x_tok.at[w_row + 1], ix_b)
        pltpu.sync_copy(y_tok.at[w_row + 1], iy_b)

        wait_gathers(0)
        start_idx_load(w_row + 2, 0)
        start_gathers(1)
        compute(0)
        start_out(w_row, 0)

        wait_gathers(1)
        start_idx_load(w_row + 3, 1)
        wait_idx_load(0)
        start_gathers(0)
        compute(1)
        start_out(w_row + 1, 1)

        def pair(j, carry):
            for p in range(2):
                row = w_row + 2 * j + p
                wait_gathers(p)
                start_idx_load(row + 2, p)
                wait_idx_load(1 - p)
                start_gathers(1 - p)
                wait_out(p)
                compute(p)
                start_out(row, p)
            return carry

        lax.fori_loop(1, n_chunks // 2 - 1, pair, 0)

        row = w_row + n_chunks - 2
        wait_gathers(0)
        wait_idx_load(1)
        start_gathers(1)
        wait_out(0)
        compute(0)
        start_out(row, 0)

        wait_gathers(1)
        wait_out(1)
        compute(1)
        start_out(row + 1, 1)

        wait_out(0)
        wait_out(1)

    return k


_sc_kernel = _make_kernel()


def kernel(x_tokens, y_tokens, x_table, y_table, pos_table):
    return _sc_kernel(x_tokens.astype(jnp.int32), y_tokens.astype(jnp.int32),
                      x_table, y_table, pos_table)


# speedup vs baseline: 1.4371x; 1.4371x over previous
import functools

import jax
import jax.numpy as jnp
from jax import lax
from jax.experimental import pallas as pl
from jax.experimental.pallas import tpu as pltpu
from jax.experimental.pallas import tpu_sc as plsc

NBINS_X = 1000
MAX_Y_TOKENS = 201
EMBED_DIM = 64
MAX_LEN = 200
B = 4096
T = 200
N = B * T

CHUNK = 200


def _make_kernel():
    info = plsc.get_sparse_core_info()
    nc, ns = info.num_cores, info.num_subcores
    nw = nc * ns
    n_chunks = B // nw
    assert B % nw == 0 and n_chunks % 2 == 0 and n_chunks >= 6

    mesh = plsc.VectorSubcoreMesh(core_axis_name="c", subcore_axis_name="s")

    f32 = jnp.float32
    i32 = jnp.int32

    @functools.partial(
        pl.kernel,
        mesh=mesh,
        out_type=jax.ShapeDtypeStruct((B, T, EMBED_DIM), f32),
        compiler_params=pltpu.CompilerParams(use_tc_tiling_on_sc=False),
        scratch_types=[
            pltpu.VMEM((CHUNK,), i32), pltpu.VMEM((CHUNK,), i32),
            pltpu.VMEM((CHUNK,), i32), pltpu.VMEM((CHUNK,), i32),
            pltpu.VMEM((CHUNK, EMBED_DIM), f32), pltpu.VMEM((CHUNK, EMBED_DIM), f32),
            pltpu.VMEM((CHUNK, EMBED_DIM), f32), pltpu.VMEM((CHUNK, EMBED_DIM), f32),
            pltpu.VMEM((CHUNK, EMBED_DIM), f32), pltpu.VMEM((CHUNK, EMBED_DIM), f32),
            pltpu.VMEM((CHUNK, EMBED_DIM), f32),
            pltpu.SemaphoreType.DMA, pltpu.SemaphoreType.DMA,
            pltpu.SemaphoreType.DMA, pltpu.SemaphoreType.DMA,
            pltpu.SemaphoreType.DMA, pltpu.SemaphoreType.DMA,
            pltpu.SemaphoreType.DMA, pltpu.SemaphoreType.DMA,
            pltpu.VMEM_SHARED((NBINS_X, EMBED_DIM), jnp.float32),
            pltpu.VMEM_SHARED((MAX_Y_TOKENS, EMBED_DIM), jnp.float32),
        ],
    )
    def k(x_tok, y_tok, x_table, y_table, pos_table, out,
          ix_a, ix_b, iy_a, iy_b, xr_a, xr_b, yr_a, yr_b, acc_a, acc_b,
          pos_v, sgx_a, sgx_b, sgy_a, sgy_b, so_a, so_b, si_a, si_b,
          xts, yts):
        wid = lax.axis_index("s") * nc + lax.axis_index("c")
        w_row = wid * n_chunks

        ix = (ix_a, ix_b)
        iy = (iy_a, iy_b)
        xr = (xr_a, xr_b)
        yr = (yr_a, yr_b)
        acc = (acc_a, acc_b)
        sgx = (sgx_a, sgx_b)
        sgy = (sgy_a, sgy_b)
        so = (so_a, so_b)
        si = (si_a, si_b)

        pltpu.sync_copy(pos_table, pos_v)

        @pl.when(lax.axis_index("s") == 0)
        def _stage():
            pltpu.sync_copy(x_table, xts)
            pltpu.sync_copy(y_table, yts)

        plsc.subcore_barrier()

        def start_gathers(p):
            pltpu.async_copy(xts.at[ix[p]], xr[p], sgx[p])
            pltpu.async_copy(yts.at[iy[p]], yr[p], sgy[p])

        def wait_gathers(p):
            pltpu.make_async_copy(xts.at[ix[p]], xr[p], sgx[p]).wait()
            pltpu.make_async_copy(yts.at[iy[p]], yr[p], sgy[p]).wait()

        def start_idx_load(row, p):
            pltpu.async_copy(x_tok.at[row], ix[p], si[p])
            pltpu.async_copy(y_tok.at[row], iy[p], si[p])

        def wait_idx_load(p):
            pltpu.make_async_copy(x_tok.at[0], ix[p], si[p]).wait()
            pltpu.make_async_copy(y_tok.at[0], iy[p], si[p]).wait()

        def compute(p):
            xp, yp, ap = xr[p], yr[p], acc[p]
            nj = EMBED_DIM // 16

            def add_row(i, c):
                xs = [xp[i, pl.ds(j * 16, 16)] for j in range(nj)]
                ys = [yp[i, pl.ds(j * 16, 16)] for j in range(nj)]
                ps = [pos_v[i, pl.ds(j * 16, 16)] for j in range(nj)]
                ss = [(xs[j] + ys[j]) + ps[j] for j in range(nj)]
                for j in range(nj):
                    ap[i, pl.ds(j * 16, 16)] = ss[j]
                return c

            lax.fori_loop(0, CHUNK, add_row, 0, unroll=2)

        def start_out(row, p):
            pltpu.async_copy(acc[p], out.at[row], so[p])

        def wait_out(p):
            pltpu.make_async_copy(acc[p], out.at[0], so[p]).wait()

        pltpu.sync_copy(x_tok.at[w_row], ix_a)
        pltpu.sync_copy(y_tok.at[w_row], iy_a)
        start_gathers(0)
        pltpu.sync_copy(x_tok.at[w_row + 1], ix_b)
        pltpu.sync_copy(y_tok.at[w_row + 1], iy_b)

        wait_gathers(0)
        start_idx_load(w_row + 2, 0)
        start_gathers(1)
        compute(0)
        start_out(w_row, 0)

        wait_gathers(1)
        start_idx_load(w_row + 3, 1)
        wait_idx_load(0)
        start_gathers(0)
        compute(1)
        start_out(w_row + 1, 1)

        def pair(j, carry):
            for p in range(2):
                row = w_row + 2 * j + p
                wait_gathers(p)
                start_idx_load(row + 2, p)
                wait_idx_load(1 - p)
                start_gathers(1 - p)
                wait_out(p)
                compute(p)
                start_out(row, p)
            return carry

        lax.fori_loop(1, n_chunks // 2 - 1, pair, 0)

        row = w_row + n_chunks - 2
        wait_gathers(0)
        wait_idx_load(1)
        start_gathers(1)
        wait_out(0)
        compute(0)
        start_out(row, 0)

        wait_gathers(1)
        wait_out(1)
        compute(1)
        start_out(row + 1, 1)

        wait_out(0)
        wait_out(1)

    return k


_sc_kernel = _make_kernel()


def kernel(x_tokens, y_tokens, x_table, y_table, pos_table):
    return _sc_kernel(x_tokens.astype(jnp.int32), y_tokens.astype(jnp.int32),
                      x_table, y_table, pos_table)
